# Initial kernel scaffold; baseline (speedup 1.0000x reference)
import jax
import jax.numpy as jnp
from jax.experimental import pallas as pl


def kernel(s_fp_list, s_bp_list, classes):
    raise NotImplementedError("write your pallas kernel here")


# speedup vs baseline: 1.0150x; 1.0150x over previous
import functools

import jax
import jax.numpy as jnp
from jax import lax
from jax.experimental import pallas as pl
from jax.experimental.pallas import tpu as pltpu
from jax.experimental.pallas import tpu_sc as plsc

NUM_CLASSES = 64
L, B, D = 4, 4096, 2048
N = L * B
NC, NS, LANES = 2, 16, 16
NSTRIPE = 16
DCOL = D // NSTRIPE
NROWG = 2
ROWS_W = N // NROWG
R = 128
NCHUNK = ROWS_W // R
DVEC = DCOL // LANES
ACC = NUM_CLASSES * DCOL
CCLS = NUM_CLASSES // NROWG
HALFACC = CCLS * DCOL


def _make_sc_kernel():
    mesh = plsc.VectorSubcoreMesh(core_axis_name="c", subcore_axis_name="s")

    @functools.partial(
        pl.kernel,
        mesh=mesh,
        compiler_params=pltpu.CompilerParams(needs_layout_passes=False),
        out_type=[
            jax.ShapeDtypeStruct((NUM_CLASSES, D), jnp.float32),
            jax.ShapeDtypeStruct((NUM_CLASSES, D), jnp.float32),
        ],
        scratch_types=[
            pltpu.VMEM_SHARED((NS, ACC), jnp.float32),
            pltpu.VMEM_SHARED((NS, ACC), jnp.float32),
            pltpu.VMEM_SHARED((NS, NUM_CLASSES * LANES), jnp.float32),
            pltpu.VMEM((ACC,), jnp.float32),
            pltpu.VMEM((ACC,), jnp.float32),
            pltpu.VMEM((NUM_CLASSES * LANES,), jnp.float32),
            pltpu.VMEM((R, DCOL), jnp.float32),
            pltpu.VMEM((R, DCOL), jnp.float32),
            pltpu.VMEM((R,), jnp.int32),
            pltpu.VMEM((HALFACC,), jnp.float32),
            pltpu.VMEM((CCLS * LANES,), jnp.float32),
            pltpu.VMEM((CCLS, DCOL), jnp.float32),
        ],
    )
    def seg_mean(fp_hbm, bp_hbm, lab_hbm, out_fp, out_bp,
                 sp_fp, sp_bp, sp_cnt, acc_fp, acc_bp, acc_cnt,
                 rowbuf_fp, rowbuf_bp, idxbuf, partbuf, cntpart, resbuf):
        cid = lax.axis_index("c")
        sid = lax.axis_index("s")
        stripe = cid * (NSTRIPE // NC) + sid // NROWG
        half = sid % NROWG
        col0 = stripe * DCOL
        row0 = half * ROWS_W
        partner = sid - half + (1 - half)

        ones_v = jnp.ones((LANES,), jnp.float32)
        zero_v = jnp.zeros((LANES,), jnp.float32)
        iota_v = lax.iota(jnp.int32, LANES)

        def zero_body(i, _):
            acc_fp[pl.ds(i * LANES, LANES)] = zero_v
            acc_bp[pl.ds(i * LANES, LANES)] = zero_v
            return 0
        lax.fori_loop(0, ACC // LANES, zero_body, 0)

        def zero_cnt_body(i, _):
            acc_cnt[pl.ds(i * LANES, LANES)] = zero_v
            return 0
        lax.fori_loop(0, NUM_CLASSES, zero_cnt_body, 0)

        def chunk_body(k, _):
            r0 = row0 + k * R
            pltpu.sync_copy(lab_hbm.at[pl.ds(r0, R)], idxbuf)
            pltpu.sync_copy(fp_hbm.at[pl.ds(r0, R), pl.ds(col0, DCOL)],
                            rowbuf_fp)
            pltpu.sync_copy(bp_hbm.at[pl.ds(r0, R), pl.ds(col0, DCOL)],
                            rowbuf_bp)

            def group_body(g, _):
                lv = idxbuf[pl.ds(g * LANES, LANES)]
                for kk in range(LANES):
                    i = g * LANES + kk
                    lab = lv[kk]
                    base = lab * DCOL + iota_v
                    cbase = lab * LANES + iota_v
                    plsc.addupdate_scatter(acc_cnt, [cbase], ones_v)
                    for j in range(DVEC):
                        plsc.addupdate_scatter(
                            acc_fp, [base + (j * LANES)],
                            rowbuf_fp[i, pl.ds(j * LANES, LANES)])
                    for j in range(DVEC):
                        plsc.addupdate_scatter(
                            acc_bp, [base + (j * LANES)],
                            rowbuf_bp[i, pl.ds(j * LANES, LANES)])
                return 0
            lax.fori_loop(0, R // LANES, group_body, 0)
            return 0
        lax.fori_loop(0, NCHUNK, chunk_body, 0)

        pltpu.sync_copy(acc_fp, sp_fp.at[sid])
        pltpu.sync_copy(acc_bp, sp_bp.at[sid])
        pltpu.sync_copy(acc_cnt, sp_cnt.at[sid])
        plsc.subcore_barrier()

        c0 = half * HALFACC
        n0 = half * (CCLS * LANES)
        pltpu.sync_copy(sp_cnt.at[partner, pl.ds(n0, CCLS * LANES)], cntpart)

        def finalize(acc, sp, out_hbm):
            pltpu.sync_copy(sp.at[partner, pl.ds(c0, HALFACC)], partbuf)
            for r in range(CCLS):
                cnt = (acc_cnt[pl.ds(n0 + r * LANES, LANES)]
                       + cntpart[pl.ds(r * LANES, LANES)])
                rec = 1.0 / jnp.maximum(cnt, 1.0)
                for j in range(DVEC):
                    o = r * DCOL + j * LANES
                    resbuf[r, pl.ds(j * LANES, LANES)] = (
                        (acc[pl.ds(c0 + o, LANES)]
                         + partbuf[pl.ds(o, LANES)]) * rec)
            pltpu.sync_copy(
                resbuf,
                out_hbm.at[pl.ds(half * CCLS, CCLS), pl.ds(col0, DCOL)])

        finalize(acc_fp, sp_fp, out_fp)
        finalize(acc_bp, sp_bp, out_bp)

    return seg_mean


_SEG_MEAN = _make_sc_kernel()


def kernel(s_fp_list, s_bp_list, classes):
    fp = s_fp_list.reshape(N, D)
    bp = s_bp_list.reshape(N, D)
    labels = classes.reshape(B, L).T.reshape(N)
    out_fp, out_bp = _SEG_MEAN(fp, bp, labels)
    return (out_fp, out_bp)


# speedup vs baseline: 1.2947x; 1.2756x over previous
import functools

import jax
import jax.numpy as jnp
from jax import lax
from jax.experimental import pallas as pl
from jax.experimental.pallas import tpu as pltpu
from jax.experimental.pallas import tpu_sc as plsc

NUM_CLASSES = 64
L, B, D = 4, 4096, 2048
N = L * B
NC, NS, LANES = 2, 16, 16
NSTRIPE = 16
DCOL = D // NSTRIPE
NROWG = 2
ROWS_W = N // NROWG
R = 128
NCHUNK = ROWS_W // R
DVEC = DCOL // LANES
ACC = NUM_CLASSES * DCOL
CCLS = NUM_CLASSES // NROWG
HALFACC = CCLS * DCOL


def _make_sc_kernel():
    mesh = plsc.VectorSubcoreMesh(core_axis_name="c", subcore_axis_name="s")

    @functools.partial(
        pl.kernel,
        mesh=mesh,
        compiler_params=pltpu.CompilerParams(needs_layout_passes=False),
        out_type=[
            jax.ShapeDtypeStruct((NUM_CLASSES, D), jnp.float32),
            jax.ShapeDtypeStruct((NUM_CLASSES, D), jnp.float32),
        ],
        scratch_types=[
            pltpu.VMEM_SHARED((NS, ACC), jnp.float32),
            pltpu.VMEM_SHARED((NS, ACC), jnp.float32),
            pltpu.VMEM_SHARED((NS, NUM_CLASSES * LANES), jnp.float32),
            pltpu.VMEM((ACC,), jnp.float32),
            pltpu.VMEM((ACC,), jnp.float32),
            pltpu.VMEM((NUM_CLASSES * LANES,), jnp.float32),
            pltpu.VMEM((2, R, DCOL), jnp.float32),
            pltpu.VMEM((2, R, DCOL), jnp.float32),
            pltpu.VMEM((2, R), jnp.int32),
            pltpu.SemaphoreType.DMA((2,)),
            pltpu.VMEM((HALFACC,), jnp.float32),
            pltpu.VMEM((CCLS * LANES,), jnp.float32),
            pltpu.VMEM((CCLS, DCOL), jnp.float32),
        ],
    )
    def seg_mean(fp_hbm, bp_hbm, lab_hbm, out_fp, out_bp,
                 sp_fp, sp_bp, sp_cnt, acc_fp, acc_bp, acc_cnt,
                 rowbuf_fp, rowbuf_bp, idxbuf, sems,
                 partbuf, cntpart, resbuf):
        cid = lax.axis_index("c")
        sid = lax.axis_index("s")
        stripe = cid * (NSTRIPE // NC) + sid // NROWG
        half = sid % NROWG
        col0 = stripe * DCOL
        row0 = half * ROWS_W
        partner = sid - half + (1 - half)

        ones_v = jnp.ones((LANES,), jnp.float32)
        zero_v = jnp.zeros((LANES,), jnp.float32)
        iota_v = lax.iota(jnp.int32, LANES)

        def zero_body(i, _):
            acc_fp[pl.ds(i * LANES, LANES)] = zero_v
            acc_bp[pl.ds(i * LANES, LANES)] = zero_v
            return 0
        lax.fori_loop(0, ACC // LANES, zero_body, 0)

        def zero_cnt_body(i, _):
            acc_cnt[pl.ds(i * LANES, LANES)] = zero_v
            return 0
        lax.fori_loop(0, NUM_CLASSES, zero_cnt_body, 0)

        def issue(b, k):
            r0 = row0 + k * R
            pltpu.async_copy(lab_hbm.at[pl.ds(r0, R)], idxbuf.at[b],
                             sems.at[b])
            pltpu.async_copy(fp_hbm.at[pl.ds(r0, R), pl.ds(col0, DCOL)],
                             rowbuf_fp.at[b], sems.at[b])
            pltpu.async_copy(bp_hbm.at[pl.ds(r0, R), pl.ds(col0, DCOL)],
                             rowbuf_bp.at[b], sems.at[b])

        def wait(b):
            pltpu.make_async_copy(lab_hbm.at[pl.ds(0, R)], idxbuf.at[b],
                                  sems.at[b]).wait()
            pltpu.make_async_copy(fp_hbm.at[pl.ds(0, R), pl.ds(0, DCOL)],
                                  rowbuf_fp.at[b], sems.at[b]).wait()
            pltpu.make_async_copy(bp_hbm.at[pl.ds(0, R), pl.ds(0, DCOL)],
                                  rowbuf_bp.at[b], sems.at[b]).wait()

        def compute(b):
            def group_body(g, _):
                lv = idxbuf[b, pl.ds(g * LANES, LANES)]
                for kk in range(LANES):
                    i = g * LANES + kk
                    lab = lv[kk]
                    base = lab * DCOL + iota_v
                    cbase = lab * LANES + iota_v
                    plsc.addupdate_scatter(acc_cnt, [cbase], ones_v)
                    for j in range(DVEC):
                        plsc.addupdate_scatter(
                            acc_fp, [base + (j * LANES)],
                            rowbuf_fp[b, i, pl.ds(j * LANES, LANES)])
                    for j in range(DVEC):
                        plsc.addupdate_scatter(
                            acc_bp, [base + (j * LANES)],
                            rowbuf_bp[b, i, pl.ds(j * LANES, LANES)])
                return 0
            lax.fori_loop(0, R // LANES, group_body, 0)

        issue(0, 0)
        issue(1, 1)

        def chunk_body(t, _):
            for b in range(2):
                k = 2 * t + b
                wait(b)
                compute(b)
                issue(b, k + 2)
            return 0
        lax.fori_loop(0, NCHUNK // 2 - 1, chunk_body, 0)
        for b in range(2):
            wait(b)
            compute(b)

        pltpu.sync_copy(acc_fp, sp_fp.at[sid])
        pltpu.sync_copy(acc_bp, sp_bp.at[sid])
        pltpu.sync_copy(acc_cnt, sp_cnt.at[sid])
        plsc.subcore_barrier()

        c0 = half * HALFACC
        n0 = half * (CCLS * LANES)
        pltpu.sync_copy(sp_cnt.at[partner, pl.ds(n0, CCLS * LANES)], cntpart)

        def finalize(acc, sp, out_hbm):
            pltpu.sync_copy(sp.at[partner, pl.ds(c0, HALFACC)], partbuf)
            for r in range(CCLS):
                cnt = (acc_cnt[pl.ds(n0 + r * LANES, LANES)]
                       + cntpart[pl.ds(r * LANES, LANES)])
                rec = 1.0 / jnp.maximum(cnt, 1.0)
                for j in range(DVEC):
                    o = r * DCOL + j * LANES
                    resbuf[r, pl.ds(j * LANES, LANES)] = (
                        (acc[pl.ds(c0 + o, LANES)]
                         + partbuf[pl.ds(o, LANES)]) * rec)
            pltpu.sync_copy(
                resbuf,
                out_hbm.at[pl.ds(half * CCLS, CCLS), pl.ds(col0, DCOL)])

        finalize(acc_fp, sp_fp, out_fp)
        finalize(acc_bp, sp_bp, out_bp)

    return seg_mean


_SEG_MEAN = _make_sc_kernel()


def kernel(s_fp_list, s_bp_list, classes):
    fp = s_fp_list.reshape(N, D)
    bp = s_bp_list.reshape(N, D)
    labels = classes.reshape(B, L).T.reshape(N)
    out_fp, out_bp = _SEG_MEAN(fp, bp, labels)
    return (out_fp, out_bp)


# speedup vs baseline: 1.3069x; 1.0094x over previous
import functools

import jax
import jax.numpy as jnp
from jax import lax
from jax.experimental import pallas as pl
from jax.experimental.pallas import tpu as pltpu
from jax.experimental.pallas import tpu_sc as plsc

NUM_CLASSES = 64
L, B, D = 4, 4096, 2048
N = L * B
NC, NS, LANES = 2, 16, 16
NSTRIPE = 16
DCOL = D // NSTRIPE
NROWG = 2
ROWS_W = N // NROWG
R = 128
NCHUNK = ROWS_W // R
DVEC = DCOL // LANES
ACC = NUM_CLASSES * DCOL
CCLS = NUM_CLASSES // NROWG
HALFACC = CCLS * DCOL


def _make_sc_kernel():
    mesh = plsc.VectorSubcoreMesh(core_axis_name="c", subcore_axis_name="s")

    @functools.partial(
        pl.kernel,
        mesh=mesh,
        compiler_params=pltpu.CompilerParams(needs_layout_passes=False),
        out_type=[
            jax.ShapeDtypeStruct((NUM_CLASSES, D), jnp.float32),
            jax.ShapeDtypeStruct((NUM_CLASSES, D), jnp.float32),
        ],
        scratch_types=[
            pltpu.VMEM_SHARED((NS, ACC), jnp.float32),
            pltpu.VMEM_SHARED((NS, ACC), jnp.float32),
            pltpu.VMEM_SHARED((NS, NUM_CLASSES * LANES), jnp.float32),
            pltpu.VMEM((ACC,), jnp.float32),
            pltpu.VMEM((ACC,), jnp.float32),
            pltpu.VMEM((NUM_CLASSES * LANES,), jnp.float32),
            pltpu.VMEM((2, R, DCOL), jnp.float32),
            pltpu.VMEM((2, R, DCOL), jnp.float32),
            pltpu.VMEM((2, R), jnp.int32),
            pltpu.SemaphoreType.DMA((2,)),
            pltpu.VMEM((HALFACC,), jnp.float32),
            pltpu.VMEM((CCLS * LANES,), jnp.float32),
            pltpu.VMEM((CCLS, DCOL), jnp.float32),
        ],
    )
    def seg_mean(fp_hbm, bp_hbm, lab_hbm, out_fp, out_bp,
                 sp_fp, sp_bp, sp_cnt, acc_fp, acc_bp, acc_cnt,
                 rowbuf_fp, rowbuf_bp, idxbuf, sems,
                 partbuf, cntpart, resbuf):
        cid = lax.axis_index("c")
        sid = lax.axis_index("s")
        stripe = cid * (NSTRIPE // NC) + sid // NROWG
        half = sid % NROWG
        col0 = stripe * DCOL
        row0 = half * ROWS_W
        partner = sid - half + (1 - half)

        ones_v = jnp.ones((LANES,), jnp.float32)
        zero_v = jnp.zeros((LANES,), jnp.float32)
        iota_v = lax.iota(jnp.int32, LANES)

        def zero_body(i, _):
            acc_fp[pl.ds(i * LANES, LANES)] = zero_v
            acc_bp[pl.ds(i * LANES, LANES)] = zero_v
            return 0
        lax.fori_loop(0, ACC // LANES, zero_body, 0)

        def zero_cnt_body(i, _):
            acc_cnt[pl.ds(i * LANES, LANES)] = zero_v
            return 0
        lax.fori_loop(0, NUM_CLASSES, zero_cnt_body, 0)

        def issue(b, k):
            r0 = row0 + k * R
            pltpu.async_copy(lab_hbm.at[pl.ds(r0, R)], idxbuf.at[b],
                             sems.at[b])
            pltpu.async_copy(fp_hbm.at[pl.ds(r0, R), pl.ds(col0, DCOL)],
                             rowbuf_fp.at[b], sems.at[b])
            pltpu.async_copy(bp_hbm.at[pl.ds(r0, R), pl.ds(col0, DCOL)],
                             rowbuf_bp.at[b], sems.at[b])

        def wait(b):
            pltpu.make_async_copy(lab_hbm.at[pl.ds(0, R)], idxbuf.at[b],
                                  sems.at[b]).wait()
            pltpu.make_async_copy(fp_hbm.at[pl.ds(0, R), pl.ds(0, DCOL)],
                                  rowbuf_fp.at[b], sems.at[b]).wait()
            pltpu.make_async_copy(bp_hbm.at[pl.ds(0, R), pl.ds(0, DCOL)],
                                  rowbuf_bp.at[b], sems.at[b]).wait()

        def compute(b):
            @plsc.parallel_loop(0, R // LANES, unroll=2)
            def group_body(g):
                lv = idxbuf[b, pl.ds(g * LANES, LANES)]
                bases = lv * DCOL
                cbases = lv * LANES
                for kk in range(LANES):
                    i = g * LANES + kk
                    base = bases[kk] + iota_v
                    cbase = cbases[kk] + iota_v
                    plsc.addupdate_scatter(acc_cnt, [cbase], ones_v)
                    for j in range(DVEC):
                        plsc.addupdate_scatter(
                            acc_fp, [base + (j * LANES)],
                            rowbuf_fp[b, i, pl.ds(j * LANES, LANES)])
                    for j in range(DVEC):
                        plsc.addupdate_scatter(
                            acc_bp, [base + (j * LANES)],
                            rowbuf_bp[b, i, pl.ds(j * LANES, LANES)])

        issue(0, 0)
        issue(1, 1)

        def chunk_body(t, _):
            for b in range(2):
                k = 2 * t + b
                wait(b)
                compute(b)
                issue(b, k + 2)
            return 0
        lax.fori_loop(0, NCHUNK // 2 - 1, chunk_body, 0)
        for b in range(2):
            wait(b)
            compute(b)

        pltpu.sync_copy(acc_fp, sp_fp.at[sid])
        pltpu.sync_copy(acc_bp, sp_bp.at[sid])
        pltpu.sync_copy(acc_cnt, sp_cnt.at[sid])
        plsc.subcore_barrier()

        c0 = half * HALFACC
        n0 = half * (CCLS * LANES)
        pltpu.sync_copy(sp_cnt.at[partner, pl.ds(n0, CCLS * LANES)], cntpart)

        def finalize(acc, sp, out_hbm):
            pltpu.sync_copy(sp.at[partner, pl.ds(c0, HALFACC)], partbuf)
            for r in range(CCLS):
                cnt = (acc_cnt[pl.ds(n0 + r * LANES, LANES)]
                       + cntpart[pl.ds(r * LANES, LANES)])
                rec = 1.0 / jnp.maximum(cnt, 1.0)
                for j in range(DVEC):
                    o = r * DCOL + j * LANES
                    resbuf[r, pl.ds(j * LANES, LANES)] = (
                        (acc[pl.ds(c0 + o, LANES)]
                         + partbuf[pl.ds(o, LANES)]) * rec)
            pltpu.sync_copy(
                resbuf,
                out_hbm.at[pl.ds(half * CCLS, CCLS), pl.ds(col0, DCOL)])

        finalize(acc_fp, sp_fp, out_fp)
        finalize(acc_bp, sp_bp, out_bp)

    return seg_mean


_SEG_MEAN = _make_sc_kernel()


def kernel(s_fp_list, s_bp_list, classes):
    fp = s_fp_list.reshape(N, D)
    bp = s_bp_list.reshape(N, D)
    labels = classes.reshape(B, L).T.reshape(N)
    out_fp, out_bp = _SEG_MEAN(fp, bp, labels)
    return (out_fp, out_bp)


# speedup vs baseline: 1.7826x; 1.3640x over previous
import functools

import jax
import jax.numpy as jnp
from jax import lax
from jax.experimental import pallas as pl
from jax.experimental.pallas import tpu as pltpu
from jax.experimental.pallas import tpu_sc as plsc

NUM_CLASSES = 64
L, B, D = 4, 4096, 2048
N = L * B
NC, NS, LANES = 2, 16, 16
NSTRIPE = 16
DCOL = D // NSTRIPE
NROWG = 2
ROWS_W = N // NROWG
R = 128
NCHUNK = ROWS_W // R
DVEC = DCOL // LANES
ACC = NUM_CLASSES * DCOL
CCLS = NUM_CLASSES // NROWG
HALFACC = CCLS * DCOL


def _make_sc_kernel():
    mesh = plsc.VectorSubcoreMesh(core_axis_name="c", subcore_axis_name="s")

    @functools.partial(
        pl.kernel,
        mesh=mesh,
        compiler_params=pltpu.CompilerParams(
            needs_layout_passes=False, use_tc_tiling_on_sc=False),
        out_type=[
            jax.ShapeDtypeStruct((NUM_CLASSES, D), jnp.float32),
            jax.ShapeDtypeStruct((NUM_CLASSES, D), jnp.float32),
        ],
        scratch_types=[
            pltpu.VMEM_SHARED((NS, ACC), jnp.float32),
            pltpu.VMEM_SHARED((NS, ACC), jnp.float32),
            pltpu.VMEM_SHARED((NS, NUM_CLASSES * LANES), jnp.float32),
            pltpu.VMEM((ACC,), jnp.float32),
            pltpu.VMEM((ACC,), jnp.float32),
            pltpu.VMEM((NUM_CLASSES * LANES,), jnp.float32),
            pltpu.VMEM((2, R, DCOL), jnp.float32),
            pltpu.VMEM((2, R, DCOL), jnp.float32),
            pltpu.VMEM((2, R), jnp.int32),
            pltpu.SemaphoreType.DMA((2,)),
            pltpu.VMEM((HALFACC,), jnp.float32),
            pltpu.VMEM((CCLS * LANES,), jnp.float32),
            pltpu.VMEM((CCLS, DCOL), jnp.float32),
        ],
    )
    def seg_mean(fp_hbm, bp_hbm, lab_hbm, out_fp, out_bp,
                 sp_fp, sp_bp, sp_cnt, acc_fp, acc_bp, acc_cnt,
                 rowbuf_fp, rowbuf_bp, idxbuf, sems,
                 partbuf, cntpart, resbuf):
        cid = lax.axis_index("c")
        sid = lax.axis_index("s")
        stripe = cid * (NSTRIPE // NC) + sid // NROWG
        half = sid % NROWG
        col0 = stripe * DCOL
        row0 = half * ROWS_W
        partner = sid - half + (1 - half)

        ones_v = jnp.ones((LANES,), jnp.float32)
        zero_v = jnp.zeros((LANES,), jnp.float32)
        iota_v = lax.iota(jnp.int32, LANES)

        def zero_body(i, _):
            acc_fp[pl.ds(i * LANES, LANES)] = zero_v
            acc_bp[pl.ds(i * LANES, LANES)] = zero_v
            return 0
        lax.fori_loop(0, ACC // LANES, zero_body, 0)

        def zero_cnt_body(i, _):
            acc_cnt[pl.ds(i * LANES, LANES)] = zero_v
            return 0
        lax.fori_loop(0, NUM_CLASSES, zero_cnt_body, 0)

        def issue(b, k):
            r0 = row0 + k * R
            pltpu.async_copy(lab_hbm.at[pl.ds(r0, R)], idxbuf.at[b],
                             sems.at[b])
            pltpu.async_copy(fp_hbm.at[pl.ds(r0, R), pl.ds(col0, DCOL)],
                             rowbuf_fp.at[b], sems.at[b])
            pltpu.async_copy(bp_hbm.at[pl.ds(r0, R), pl.ds(col0, DCOL)],
                             rowbuf_bp.at[b], sems.at[b])

        def wait(b):
            pltpu.make_async_copy(lab_hbm.at[pl.ds(0, R)], idxbuf.at[b],
                                  sems.at[b]).wait()
            pltpu.make_async_copy(fp_hbm.at[pl.ds(0, R), pl.ds(0, DCOL)],
                                  rowbuf_fp.at[b], sems.at[b]).wait()
            pltpu.make_async_copy(bp_hbm.at[pl.ds(0, R), pl.ds(0, DCOL)],
                                  rowbuf_bp.at[b], sems.at[b]).wait()

        def compute(b):
            @plsc.parallel_loop(0, R // LANES, unroll=2)
            def group_body(g):
                lv = idxbuf[b, pl.ds(g * LANES, LANES)]
                bases = lv * DCOL
                cbases = lv * LANES
                for kk in range(LANES):
                    i = g * LANES + kk
                    base = bases[kk] + iota_v
                    cbase = cbases[kk] + iota_v
                    plsc.addupdate_scatter(acc_cnt, [cbase], ones_v)
                    for j in range(DVEC):
                        plsc.addupdate_scatter(
                            acc_fp, [base + (j * LANES)],
                            rowbuf_fp[b, i, pl.ds(j * LANES, LANES)])
                    for j in range(DVEC):
                        plsc.addupdate_scatter(
                            acc_bp, [base + (j * LANES)],
                            rowbuf_bp[b, i, pl.ds(j * LANES, LANES)])

        issue(0, 0)
        issue(1, 1)

        def chunk_body(t, _):
            for b in range(2):
                k = 2 * t + b
                wait(b)
                compute(b)
                issue(b, k + 2)
            return 0
        lax.fori_loop(0, NCHUNK // 2 - 1, chunk_body, 0)
        for b in range(2):
            wait(b)
            compute(b)

        pltpu.sync_copy(acc_fp, sp_fp.at[sid])
        pltpu.sync_copy(acc_bp, sp_bp.at[sid])
        pltpu.sync_copy(acc_cnt, sp_cnt.at[sid])
        plsc.subcore_barrier()

        c0 = half * HALFACC
        n0 = half * (CCLS * LANES)
        pltpu.sync_copy(sp_cnt.at[partner, pl.ds(n0, CCLS * LANES)], cntpart)

        def finalize(acc, sp, out_hbm):
            pltpu.sync_copy(sp.at[partner, pl.ds(c0, HALFACC)], partbuf)
            for r in range(CCLS):
                cnt = (acc_cnt[pl.ds(n0 + r * LANES, LANES)]
                       + cntpart[pl.ds(r * LANES, LANES)])
                rec = 1.0 / jnp.maximum(cnt, 1.0)
                for j in range(DVEC):
                    o = r * DCOL + j * LANES
                    resbuf[r, pl.ds(j * LANES, LANES)] = (
                        (acc[pl.ds(c0 + o, LANES)]
                         + partbuf[pl.ds(o, LANES)]) * rec)
            pltpu.sync_copy(
                resbuf,
                out_hbm.at[pl.ds(half * CCLS, CCLS), pl.ds(col0, DCOL)])

        finalize(acc_fp, sp_fp, out_fp)
        finalize(acc_bp, sp_bp, out_bp)

    return seg_mean


_SEG_MEAN = _make_sc_kernel()


def kernel(s_fp_list, s_bp_list, classes):
    fp = s_fp_list.reshape(N, D)
    bp = s_bp_list.reshape(N, D)
    labels = classes.reshape(B, L).T.reshape(N)
    out_fp, out_bp = _SEG_MEAN(fp, bp, labels)
    return (out_fp, out_bp)


# speedup vs baseline: 1.8166x; 1.0191x over previous
import functools

import jax
import jax.numpy as jnp
from jax import lax
from jax.experimental import pallas as pl
from jax.experimental.pallas import tpu as pltpu
from jax.experimental.pallas import tpu_sc as plsc

NUM_CLASSES = 64
L, B, D = 4, 4096, 2048
N = L * B
NC, NS, LANES = 2, 16, 16
NSTRIPE = 16
DCOL = D // NSTRIPE
NROWG = 2
ROWS_W = N // NROWG
R = 128
NCHUNK = ROWS_W // R
DVEC = DCOL // LANES
ACC = NUM_CLASSES * DCOL
CCLS = NUM_CLASSES // NROWG
HALFACC = CCLS * DCOL


def _make_sc_kernel():
    mesh = plsc.VectorSubcoreMesh(core_axis_name="c", subcore_axis_name="s")

    @functools.partial(
        pl.kernel,
        mesh=mesh,
        compiler_params=pltpu.CompilerParams(
            needs_layout_passes=False, use_tc_tiling_on_sc=False),
        out_type=[
            jax.ShapeDtypeStruct((NUM_CLASSES, D), jnp.float32),
            jax.ShapeDtypeStruct((NUM_CLASSES, D), jnp.float32),
        ],
        scratch_types=[
            pltpu.VMEM_SHARED((NS, ACC), jnp.float32),
            pltpu.VMEM_SHARED((NS, ACC), jnp.float32),
            pltpu.VMEM_SHARED((NS, NUM_CLASSES * LANES), jnp.float32),
            pltpu.VMEM((ACC,), jnp.float32),
            pltpu.VMEM((ACC,), jnp.float32),
            pltpu.VMEM((NUM_CLASSES * LANES,), jnp.float32),
            pltpu.VMEM((2, R, DCOL), jnp.float32),
            pltpu.VMEM((2, R, DCOL), jnp.float32),
            pltpu.VMEM((2, R), jnp.int32),
            pltpu.SemaphoreType.DMA((2,)),
            pltpu.VMEM((HALFACC,), jnp.float32),
            pltpu.VMEM((CCLS * LANES,), jnp.float32),
            pltpu.VMEM((CCLS, DCOL), jnp.float32),
        ],
    )
    def seg_mean(fp_hbm, bp_hbm, lab_hbm, out_fp, out_bp,
                 sp_fp, sp_bp, sp_cnt, acc_fp, acc_bp, acc_cnt,
                 rowbuf_fp, rowbuf_bp, idxbuf, sems,
                 partbuf, cntpart, resbuf):
        cid = lax.axis_index("c")
        sid = lax.axis_index("s")
        stripe = cid * (NSTRIPE // NC) + sid // NROWG
        half = sid % NROWG
        col0 = stripe * DCOL
        row0 = half * ROWS_W
        partner = sid - half + (1 - half)

        ones_v = jnp.ones((LANES,), jnp.float32)
        zero_v = jnp.zeros((LANES,), jnp.float32)
        iota_v = lax.iota(jnp.int32, LANES)

        def zero_body(i, _):
            acc_fp[pl.ds(i * LANES, LANES)] = zero_v
            acc_bp[pl.ds(i * LANES, LANES)] = zero_v
            return 0
        lax.fori_loop(0, ACC // LANES, zero_body, 0)

        def zero_cnt_body(i, _):
            acc_cnt[pl.ds(i * LANES, LANES)] = zero_v
            return 0
        lax.fori_loop(0, NUM_CLASSES, zero_cnt_body, 0)

        def issue(b, k):
            r0 = row0 + k * R
            pltpu.async_copy(lab_hbm.at[pl.ds(r0, R)], idxbuf.at[b],
                             sems.at[b])
            pltpu.async_copy(fp_hbm.at[pl.ds(r0, R), pl.ds(col0, DCOL)],
                             rowbuf_fp.at[b], sems.at[b])
            pltpu.async_copy(bp_hbm.at[pl.ds(r0, R), pl.ds(col0, DCOL)],
                             rowbuf_bp.at[b], sems.at[b])

        def wait(b):
            pltpu.make_async_copy(lab_hbm.at[pl.ds(0, R)], idxbuf.at[b],
                                  sems.at[b]).wait()
            pltpu.make_async_copy(fp_hbm.at[pl.ds(0, R), pl.ds(0, DCOL)],
                                  rowbuf_fp.at[b], sems.at[b]).wait()
            pltpu.make_async_copy(bp_hbm.at[pl.ds(0, R), pl.ds(0, DCOL)],
                                  rowbuf_bp.at[b], sems.at[b]).wait()

        def compute(b):
            @plsc.parallel_loop(0, R // LANES, unroll=2)
            def group_body(g):
                lv = idxbuf[b, pl.ds(g * LANES, LANES)]
                plsc.addupdate_scatter(acc_cnt, [lv * LANES + iota_v],
                                       ones_v)
                for kk in range(LANES):
                    i = g * LANES + kk
                    off = lv[kk] * DCOL
                    for j in range(DVEC):
                        plsc.addupdate(
                            acc_fp.at[pl.ds(off + (j * LANES), LANES)],
                            rowbuf_fp[b, i, pl.ds(j * LANES, LANES)])
                    for j in range(DVEC):
                        plsc.addupdate(
                            acc_bp.at[pl.ds(off + (j * LANES), LANES)],
                            rowbuf_bp[b, i, pl.ds(j * LANES, LANES)])

        issue(0, 0)
        issue(1, 1)

        def chunk_body(t, _):
            for b in range(2):
                k = 2 * t + b
                wait(b)
                compute(b)
                issue(b, k + 2)
            return 0
        lax.fori_loop(0, NCHUNK // 2 - 1, chunk_body, 0)
        for b in range(2):
            wait(b)
            compute(b)

        pltpu.sync_copy(acc_fp, sp_fp.at[sid])
        pltpu.sync_copy(acc_bp, sp_bp.at[sid])
        pltpu.sync_copy(acc_cnt, sp_cnt.at[sid])
        plsc.subcore_barrier()

        c0 = half * HALFACC
        n0 = half * (CCLS * LANES)
        pltpu.sync_copy(sp_cnt.at[partner, pl.ds(n0, CCLS * LANES)], cntpart)

        def finalize(acc, sp, out_hbm):
            pltpu.sync_copy(sp.at[partner, pl.ds(c0, HALFACC)], partbuf)
            for r in range(CCLS):
                cnt = (acc_cnt[pl.ds(n0 + r * LANES, LANES)]
                       + cntpart[pl.ds(r * LANES, LANES)])
                tot = jnp.broadcast_to(jnp.sum(cnt), (LANES,))
                rec = 1.0 / jnp.maximum(tot, 1.0)
                for j in range(DVEC):
                    o = r * DCOL + j * LANES
                    resbuf[r, pl.ds(j * LANES, LANES)] = (
                        (acc[pl.ds(c0 + o, LANES)]
                         + partbuf[pl.ds(o, LANES)]) * rec)
            pltpu.sync_copy(
                resbuf,
                out_hbm.at[pl.ds(half * CCLS, CCLS), pl.ds(col0, DCOL)])

        finalize(acc_fp, sp_fp, out_fp)
        finalize(acc_bp, sp_bp, out_bp)

    return seg_mean


_SEG_MEAN = _make_sc_kernel()


def kernel(s_fp_list, s_bp_list, classes):
    fp = s_fp_list.reshape(N, D)
    bp = s_bp_list.reshape(N, D)
    labels = classes.reshape(B, L).T.reshape(N)
    out_fp, out_bp = _SEG_MEAN(fp, bp, labels)
    return (out_fp, out_bp)


# speedup vs baseline: 3.5936x; 1.9782x over previous
import functools

import jax
import jax.numpy as jnp
from jax import lax
from jax.experimental import pallas as pl
from jax.experimental.pallas import tpu as pltpu
from jax.experimental.pallas import tpu_sc as plsc

NUM_CLASSES = 64
L, B, D = 4, 4096, 2048
N = L * B
NC, NS, LANES = 2, 16, 16
NSTRIPE = 16
DCOL = D // NSTRIPE
NROWG = 2
ROWS_W = N // NROWG
R = 128
NCHUNK = ROWS_W // R
DVEC = DCOL // LANES
ACC = NUM_CLASSES * DCOL
CCLS = NUM_CLASSES // NROWG
HALFACC = CCLS * DCOL


def _make_sc_kernel():
    mesh = plsc.VectorSubcoreMesh(core_axis_name="c", subcore_axis_name="s")

    @functools.partial(
        pl.kernel,
        mesh=mesh,
        compiler_params=pltpu.CompilerParams(
            needs_layout_passes=False, use_tc_tiling_on_sc=False),
        out_type=[
            jax.ShapeDtypeStruct((NUM_CLASSES, D), jnp.float32),
            jax.ShapeDtypeStruct((NUM_CLASSES, D), jnp.float32),
        ],
        scratch_types=[
            pltpu.VMEM_SHARED((NS, ACC), jnp.float32),
            pltpu.VMEM_SHARED((NS, ACC), jnp.float32),
            pltpu.VMEM_SHARED((NS, NUM_CLASSES * LANES), jnp.float32),
            pltpu.VMEM((ACC,), jnp.float32),
            pltpu.VMEM((ACC,), jnp.float32),
            pltpu.VMEM((NUM_CLASSES * LANES,), jnp.float32),
            pltpu.VMEM((2, R, DCOL), jnp.float32),
            pltpu.VMEM((2, R, DCOL), jnp.float32),
            pltpu.VMEM((2, R), jnp.int32),
            pltpu.SemaphoreType.DMA((2,)),
            pltpu.VMEM((HALFACC,), jnp.float32),
            pltpu.VMEM((CCLS * LANES,), jnp.float32),
            pltpu.VMEM((CCLS, DCOL), jnp.float32),
        ],
    )
    def seg_mean(fp_hbm, bp_hbm, lab_hbm, out_fp, out_bp,
                 sp_fp, sp_bp, sp_cnt, acc_fp, acc_bp, acc_cnt,
                 rowbuf_fp, rowbuf_bp, idxbuf, sems,
                 partbuf, cntpart, resbuf):
        cid = lax.axis_index("c")
        sid = lax.axis_index("s")
        stripe = cid * (NSTRIPE // NC) + sid // NROWG
        half = sid % NROWG
        col0 = stripe * DCOL
        row0 = half * ROWS_W
        partner = sid - half + (1 - half)

        ones_v = jnp.ones((LANES,), jnp.float32)
        zero_v = jnp.zeros((LANES,), jnp.float32)
        iota_v = lax.iota(jnp.int32, LANES)

        def zero_body(i, _):
            acc_fp[pl.ds(i * LANES, LANES)] = zero_v
            acc_bp[pl.ds(i * LANES, LANES)] = zero_v
            return 0
        lax.fori_loop(0, ACC // LANES, zero_body, 0)

        def zero_cnt_body(i, _):
            acc_cnt[pl.ds(i * LANES, LANES)] = zero_v
            return 0
        lax.fori_loop(0, NUM_CLASSES, zero_cnt_body, 0)

        def issue(b, k):
            r0 = row0 + k * R
            pltpu.async_copy(lab_hbm.at[pl.ds(r0, R)], idxbuf.at[b],
                             sems.at[b])
            pltpu.async_copy(fp_hbm.at[pl.ds(r0, R), pl.ds(col0, DCOL)],
                             rowbuf_fp.at[b], sems.at[b])
            pltpu.async_copy(bp_hbm.at[pl.ds(r0, R), pl.ds(col0, DCOL)],
                             rowbuf_bp.at[b], sems.at[b])

        def wait(b):
            pltpu.make_async_copy(lab_hbm.at[pl.ds(0, R)], idxbuf.at[b],
                                  sems.at[b]).wait()
            pltpu.make_async_copy(fp_hbm.at[pl.ds(0, R), pl.ds(0, DCOL)],
                                  rowbuf_fp.at[b], sems.at[b]).wait()
            pltpu.make_async_copy(bp_hbm.at[pl.ds(0, R), pl.ds(0, DCOL)],
                                  rowbuf_bp.at[b], sems.at[b]).wait()

        def compute(b):
            @plsc.parallel_loop(0, R // LANES, unroll=2)
            def group_body(g):
                lv = idxbuf[b, pl.ds(g * LANES, LANES)]
                plsc.addupdate_scatter(acc_cnt, [lv * LANES + iota_v],
                                       ones_v)
                for kk in range(LANES):
                    i = g * LANES + kk
                    off = lv[kk] * DCOL
                    fvals = [rowbuf_fp[b, i, pl.ds(j * LANES, LANES)]
                             for j in range(DVEC)]
                    bvals = [rowbuf_bp[b, i, pl.ds(j * LANES, LANES)]
                             for j in range(DVEC)]
                    for j in range(DVEC):
                        plsc.addupdate(
                            acc_fp.at[pl.ds(off + (j * LANES), LANES)],
                            fvals[j])
                    for j in range(DVEC):
                        plsc.addupdate(
                            acc_bp.at[pl.ds(off + (j * LANES), LANES)],
                            bvals[j])

        issue(0, 0)
        issue(1, 1)

        def chunk_body(t, _):
            for b in range(2):
                k = 2 * t + b
                wait(b)
                compute(b)
                issue(b, k + 2)
            return 0
        lax.fori_loop(0, NCHUNK // 2 - 1, chunk_body, 0)
        for b in range(2):
            wait(b)
            compute(b)

        pltpu.sync_copy(acc_fp, sp_fp.at[sid])
        pltpu.sync_copy(acc_bp, sp_bp.at[sid])
        pltpu.sync_copy(acc_cnt, sp_cnt.at[sid])
        plsc.subcore_barrier()

        c0 = half * HALFACC
        n0 = half * (CCLS * LANES)
        pltpu.sync_copy(sp_cnt.at[partner, pl.ds(n0, CCLS * LANES)], cntpart)

        def finalize(acc, sp, out_hbm):
            pltpu.sync_copy(sp.at[partner, pl.ds(c0, HALFACC)], partbuf)
            for r in range(CCLS):
                cnt = (acc_cnt[pl.ds(n0 + r * LANES, LANES)]
                       + cntpart[pl.ds(r * LANES, LANES)])
                tot = jnp.broadcast_to(jnp.sum(cnt), (LANES,))
                rec = 1.0 / jnp.maximum(tot, 1.0)
                for j in range(DVEC):
                    o = r * DCOL + j * LANES
                    resbuf[r, pl.ds(j * LANES, LANES)] = (
                        (acc[pl.ds(c0 + o, LANES)]
                         + partbuf[pl.ds(o, LANES)]) * rec)
            pltpu.sync_copy(
                resbuf,
                out_hbm.at[pl.ds(half * CCLS, CCLS), pl.ds(col0, DCOL)])

        finalize(acc_fp, sp_fp, out_fp)
        finalize(acc_bp, sp_bp, out_bp)

    return seg_mean


_SEG_MEAN = _make_sc_kernel()


def kernel(s_fp_list, s_bp_list, classes):
    fp = s_fp_list.reshape(N, D)
    bp = s_bp_list.reshape(N, D)
    labels = classes.reshape(B, L).T.reshape(N)
    out_fp, out_bp = _SEG_MEAN(fp, bp, labels)
    return (out_fp, out_bp)
